# Initial kernel scaffold; baseline (speedup 1.0000x reference)
import jax
import jax.numpy as jnp
from jax.experimental import pallas as pl


def kernel(node_scalar, edge_index, W_in, b_in, W_self, b_self, W_nei, b_nei, W_out, b_out):
    raise NotImplementedError("write your pallas kernel here")


# speedup vs baseline: 6.1689x; 6.1689x over previous
import functools

import jax
import jax.numpy as jnp
from jax import lax
from jax.experimental import pallas as pl
from jax.experimental.pallas import tpu as pltpu
from jax.experimental.pallas import tpu_sc as plsc

NUM_CORES = 2
NUM_SUBCORES = 16
NUM_TILES = NUM_CORES * NUM_SUBCORES
CH = 100
GRP = 4
ZB = 32


def _pre_body(ns_ref, w_in_ref, b_in_ref, w_self_ref, b_self_ref,
              w_nei_ref, b_nei_ref, u_ref, v_ref):
    x = jnp.dot(ns_ref[...], w_in_ref[...], preferred_element_type=jnp.float32)
    x = jnp.maximum(x + b_in_ref[...], 0.0)
    u = jnp.dot(x, w_self_ref[...], preferred_element_type=jnp.float32)
    u_ref[...] = u + b_self_ref[...] + b_nei_ref[...]
    v_ref[...] = jnp.dot(x, w_nei_ref[...], preferred_element_type=jnp.float32)


def _post_body(u_ref, s_ref, deg_ref, w_out_ref, b_out_ref, out_ref, acc_ref,
               *, inv_n):
    j = pl.program_id(0)

    @pl.when(j == 0)
    def _():
        acc_ref[...] = jnp.zeros_like(acc_ref)

    s = s_ref[0] + s_ref[1]
    deg = deg_ref[0] + deg_ref[1]
    agg = s / jnp.maximum(deg, 1.0)
    hh = jnp.maximum(u_ref[...] + agg, 0.0)
    acc_ref[...] += jnp.sum(hh, axis=0, keepdims=True)

    @pl.when(j == pl.num_programs(0) - 1)
    def _():
        g = acc_ref[...] * inv_n
        out_ref[...] = (
            jnp.dot(g, w_out_ref[...], preferred_element_type=jnp.float32)
            + b_out_ref[...])


def _make_sc_segsum(npad, h, nch):
    prow = npad // NUM_SUBCORES
    assert prow % ZB == 0
    mesh = plsc.VectorSubcoreMesh(
        core_axis_name="c", subcore_axis_name="s",
        num_cores=NUM_CORES, num_subcores=NUM_SUBCORES)

    @functools.partial(
        pl.kernel, mesh=mesh,
        out_type=[
            jax.ShapeDtypeStruct((NUM_CORES, npad, h), jnp.float32),
            jax.ShapeDtypeStruct((NUM_CORES, npad, h), jnp.float32),
        ],
        scratch_types=[
            pltpu.VMEM_SHARED((npad, h), jnp.float32),
            pltpu.VMEM((GRP, CH), jnp.int32),
            pltpu.VMEM((GRP, CH), jnp.int32),
            pltpu.VMEM((CH, h), jnp.float32),
            pltpu.VMEM((ZB, h), jnp.float32),
            pltpu.SemaphoreType.DMA,
        ],
    )
    def sc_segsum(v_hbm, src_hbm, dst_hbm, zrows_hbm, ones_hbm,
                  s_out, deg_out, accum, src_v, dst_v, rows_v, zbuf, sem):
        c = lax.axis_index("c")
        s = lax.axis_index("s")
        wid = c * NUM_SUBCORES + s
        base = s * prow

        def zero_slice():
            pltpu.sync_copy(zrows_hbm, zbuf)

            @pl.loop(0, prow // ZB)
            def zblk(k):
                pltpu.sync_copy(zbuf, accum.at[pl.ds(base + k * ZB, ZB)])

        def copy_out(dst_hbm_ref):
            @pl.loop(0, prow // ZB)
            def oblk(k):
                pltpu.sync_copy(accum.at[pl.ds(base + k * ZB, ZB)], zbuf)
                pltpu.sync_copy(zbuf, dst_hbm_ref.at[c, pl.ds(base + k * ZB,
                                                              ZB)])

        zero_slice()
        plsc.subcore_barrier()

        @pl.loop(0, nch // GRP)
        def group(g):
            pltpu.sync_copy(src_hbm.at[wid, g], src_v)
            pltpu.sync_copy(dst_hbm.at[wid, g], dst_v)

            @pl.loop(0, GRP)
            def body(r):
                pltpu.async_copy(v_hbm.at[src_v.at[r]], rows_v, sem).wait()
                pltpu.sync_copy(rows_v, accum.at[dst_v.at[r]], add=True)

        plsc.subcore_barrier()
        copy_out(s_out)

        zero_slice()
        pltpu.sync_copy(ones_hbm, rows_v)
        plsc.subcore_barrier()

        @pl.loop(0, nch // GRP)
        def group_b(g):
            pltpu.sync_copy(dst_hbm.at[wid, g], dst_v)

            @pl.loop(0, GRP)
            def body_b(r):
                pltpu.sync_copy(rows_v, accum.at[dst_v.at[r]], add=True)

        plsc.subcore_barrier()
        copy_out(deg_out)

    return sc_segsum


def kernel(node_scalar, edge_index, W_in, b_in, W_self, b_self, W_nei, b_nei,
           W_out, b_out):
    n, fin = node_scalar.shape
    h = W_in.shape[1]
    out_dim = W_out.shape[1]
    e = edge_index.shape[1]
    assert e % (NUM_TILES * CH) == 0
    nch = e // (NUM_TILES * CH)
    step = NUM_SUBCORES * ZB
    npad = (n + step - 1) // step * step

    nb = 2000
    u, v = pl.pallas_call(
        _pre_body,
        grid=(n // nb,),
        in_specs=[
            pl.BlockSpec((nb, fin), lambda i: (i, 0)),
            pl.BlockSpec((fin, h), lambda i: (0, 0)),
            pl.BlockSpec((1, h), lambda i: (0, 0)),
            pl.BlockSpec((h, h), lambda i: (0, 0)),
            pl.BlockSpec((1, h), lambda i: (0, 0)),
            pl.BlockSpec((h, h), lambda i: (0, 0)),
            pl.BlockSpec((1, h), lambda i: (0, 0)),
        ],
        out_specs=[
            pl.BlockSpec((nb, h), lambda i: (i, 0)),
            pl.BlockSpec((nb, h), lambda i: (i, 0)),
        ],
        out_shape=[
            jax.ShapeDtypeStruct((n, h), jnp.float32),
            jax.ShapeDtypeStruct((n, h), jnp.float32),
        ],
    )(node_scalar, W_in, b_in.reshape(1, h), W_self, b_self.reshape(1, h),
      W_nei, b_nei.reshape(1, h))

    assert nch % GRP == 0
    src3 = edge_index[0].reshape(NUM_TILES, nch // GRP, GRP, CH)
    dst3 = edge_index[1].reshape(NUM_TILES, nch // GRP, GRP, CH)
    zrows = jnp.zeros((ZB, h), jnp.float32)
    ones = jnp.ones((CH, h), jnp.float32)
    s_part, deg_part = _make_sc_segsum(npad, h, nch)(
        v, src3, dst3, zrows, ones)

    s_part = s_part[:, :n]
    deg_col = deg_part[:, :n, 0:1]
    out = pl.pallas_call(
        functools.partial(_post_body, inv_n=1.0 / n),
        grid=(n // nb,),
        in_specs=[
            pl.BlockSpec((nb, h), lambda i: (i, 0)),
            pl.BlockSpec((NUM_CORES, nb, h), lambda i: (0, i, 0)),
            pl.BlockSpec((NUM_CORES, nb, 1), lambda i: (0, i, 0)),
            pl.BlockSpec((h, out_dim), lambda i: (0, 0)),
            pl.BlockSpec((1, out_dim), lambda i: (0, 0)),
        ],
        out_specs=pl.BlockSpec((1, out_dim), lambda i: (0, 0)),
        out_shape=jax.ShapeDtypeStruct((1, out_dim), jnp.float32),
        scratch_shapes=[pltpu.VMEM((1, h), jnp.float32)],
    )(u, s_part, deg_col, W_out, b_out.reshape(1, out_dim))

    return out.reshape(out_dim)


# speedup vs baseline: 7.8469x; 1.2720x over previous
import functools

import jax
import jax.numpy as jnp
from jax import lax
from jax.experimental import pallas as pl
from jax.experimental.pallas import tpu as pltpu
from jax.experimental.pallas import tpu_sc as plsc

NUM_CORES = 2
NUM_SUBCORES = 16
NUM_TILES = NUM_CORES * NUM_SUBCORES
CH = 100
GRP = 10
ZB = 8
HR = 80
DBLK = 8


def _pre_body(ns_ref, w_in_ref, b_in_ref, w_self_ref, b_self_ref,
              w_nei_ref, b_nei_ref, u_ref, v_ref):
    x = jnp.dot(ns_ref[...], w_in_ref[...], preferred_element_type=jnp.float32)
    x = jnp.maximum(x + b_in_ref[...], 0.0)
    u = jnp.dot(x, w_self_ref[...], preferred_element_type=jnp.float32)
    u_ref[...] = u + b_self_ref[...] + b_nei_ref[...]
    v_ref[...] = jnp.dot(x, w_nei_ref[...], preferred_element_type=jnp.float32)


def _post_body(u_ref, s_ref, deg_ref, w_out_ref, b_out_ref, out_ref, acc_ref,
               *, inv_n):
    j = pl.program_id(0)

    @pl.when(j == 0)
    def _():
        acc_ref[...] = jnp.zeros_like(acc_ref)

    s = s_ref[0] + s_ref[1]
    agg = s / jnp.maximum(deg_ref[...], 1.0)
    hh = jnp.maximum(u_ref[...] + agg, 0.0)
    acc_ref[...] += jnp.sum(hh, axis=0, keepdims=True)

    @pl.when(j == pl.num_programs(0) - 1)
    def _():
        g = acc_ref[...] * inv_n
        out_ref[...] = (
            jnp.dot(g, w_out_ref[...], preferred_element_type=jnp.float32)
            + b_out_ref[...])


def _make_sc_segsum(npad, h, nch, ndst):
    prow = npad // NUM_SUBCORES
    assert prow % ZB == 0 and npad == HR * h
    mesh = plsc.VectorSubcoreMesh(
        core_axis_name="c", subcore_axis_name="s",
        num_cores=NUM_CORES, num_subcores=NUM_SUBCORES)

    @functools.partial(
        pl.kernel, mesh=mesh,
        compiler_params=pltpu.CompilerParams(needs_layout_passes=False),
        out_type=[
            jax.ShapeDtypeStruct((NUM_CORES, npad, h), jnp.float32),
            jax.ShapeDtypeStruct((NUM_CORES, NUM_SUBCORES, npad), jnp.float32),
        ],
        scratch_types=[
            pltpu.VMEM_SHARED((npad, h), jnp.float32),
            pltpu.VMEM((GRP, CH), jnp.int32),
            pltpu.VMEM((GRP, CH), jnp.int32),
            pltpu.VMEM((2, CH, h), jnp.float32),
            pltpu.VMEM((ZB, h), jnp.float32),
            pltpu.VMEM((npad,), jnp.float32),
            pltpu.VMEM((DBLK, 16), jnp.int32),
            pltpu.SemaphoreType.DMA,
            pltpu.SemaphoreType.DMA,
        ],
    )
    def sc_segsum(v_hbm, src_hbm, dst_hbm, dst16_hbm, zrows_hbm, zflat_hbm,
                  s_out, deg_out, accum, src_v, dst_v, rows2, zbuf,
                  hist, dst16_v, sem0, sem1):
        c = lax.axis_index("c")
        s = lax.axis_index("s")
        wid = c * NUM_SUBCORES + s
        base = s * prow
        sems = (sem0, sem1)

        pltpu.sync_copy(zrows_hbm, zbuf)

        @pl.loop(0, prow // ZB)
        def zblk(k):
            pltpu.sync_copy(zbuf, accum.at[pl.ds(base + k * ZB, ZB)])

        zf = zflat_hbm.shape[0]

        @pl.loop(0, npad // zf)
        def zhist(k):
            pltpu.sync_copy(zflat_hbm, hist.at[pl.ds(k * zf, zf)])

        plsc.subcore_barrier()

        @pl.loop(0, nch // GRP)
        def group(g):
            pltpu.sync_copy(src_hbm.at[wid, g], src_v)
            pltpu.sync_copy(dst_hbm.at[wid, g], dst_v)
            descs = [None, None]
            descs[0] = pltpu.async_copy(
                v_hbm.at[src_v.at[0]], rows2.at[0], sems[0])
            for r in range(GRP):
                b = r % 2
                descs[b].wait()
                if r + 1 < GRP:
                    nb = (r + 1) % 2
                    descs[nb] = pltpu.async_copy(
                        v_hbm.at[src_v.at[r + 1]], rows2.at[nb], sems[nb])
                pltpu.sync_copy(rows2.at[b], accum.at[dst_v.at[r]], add=True)

        ones16 = jnp.ones((16,), jnp.float32)

        @pl.loop(0, ndst // DBLK)
        def hblk(t):
            pltpu.sync_copy(dst16_hbm.at[wid, pl.ds(t * DBLK, DBLK)], dst16_v)
            for i in range(DBLK):
                plsc.addupdate_scatter(hist, [dst16_v[i]], ones16)

        plsc.subcore_barrier()

        @pl.loop(0, prow // ZB)
        def oblk(k):
            pltpu.sync_copy(accum.at[pl.ds(base + k * ZB, ZB)], zbuf)
            pltpu.sync_copy(zbuf, s_out.at[c, pl.ds(base + k * ZB, ZB)])

        pltpu.sync_copy(hist, deg_out.at[c, s])

    return sc_segsum


def kernel(node_scalar, edge_index, W_in, b_in, W_self, b_self, W_nei, b_nei,
           W_out, b_out):
    n, fin = node_scalar.shape
    h = W_in.shape[1]
    out_dim = W_out.shape[1]
    e = edge_index.shape[1]
    assert e % (NUM_TILES * CH) == 0
    nch = e // (NUM_TILES * CH)
    npad = HR * h
    assert npad >= n

    nb = 2000
    u, v = pl.pallas_call(
        _pre_body,
        grid=(n // nb,),
        in_specs=[
            pl.BlockSpec((nb, fin), lambda i: (i, 0)),
            pl.BlockSpec((fin, h), lambda i: (0, 0)),
            pl.BlockSpec((1, h), lambda i: (0, 0)),
            pl.BlockSpec((h, h), lambda i: (0, 0)),
            pl.BlockSpec((1, h), lambda i: (0, 0)),
            pl.BlockSpec((h, h), lambda i: (0, 0)),
            pl.BlockSpec((1, h), lambda i: (0, 0)),
        ],
        out_specs=[
            pl.BlockSpec((nb, h), lambda i: (i, 0)),
            pl.BlockSpec((nb, h), lambda i: (i, 0)),
        ],
        out_shape=[
            jax.ShapeDtypeStruct((n, h), jnp.float32),
            jax.ShapeDtypeStruct((n, h), jnp.float32),
        ],
    )(node_scalar, W_in, b_in.reshape(1, h), W_self, b_self.reshape(1, h),
      W_nei, b_nei.reshape(1, h))

    assert nch % GRP == 0
    src3 = edge_index[0].reshape(NUM_TILES, nch // GRP, GRP, CH)
    dst3 = edge_index[1].reshape(NUM_TILES, nch // GRP, GRP, CH)
    ept = -(-e // (NUM_TILES * DBLK * 16)) * DBLK * 16
    ndst = ept // 16
    pad = NUM_TILES * ept - e
    dst16 = jnp.concatenate(
        [edge_index[1], jnp.full((pad,), npad - 1, jnp.int32)]
    ).reshape(NUM_TILES, ndst, 16)
    zrows = jnp.zeros((ZB, h), jnp.float32)
    zflat = jnp.zeros((2048,), jnp.float32)
    s_part, deg_part = _make_sc_segsum(npad, h, nch, ndst)(
        v, src3, dst3, dst16, zrows, zflat)

    dsum = pl.pallas_call(
        lambda d_ref, o_ref: o_ref.__setitem__(
            (Ellipsis,), jnp.sum(d_ref[...], axis=0)),
        in_specs=[pl.BlockSpec((NUM_TILES, HR, h), lambda: (0, 0, 0))],
        out_specs=pl.BlockSpec((HR, h), lambda: (0, 0)),
        out_shape=jax.ShapeDtypeStruct((HR, h), jnp.float32),
    )(deg_part.reshape(NUM_TILES, HR, h))
    deg_col = dsum.reshape(npad, 1)

    out = pl.pallas_call(
        functools.partial(_post_body, inv_n=1.0 / n),
        grid=(n // nb,),
        in_specs=[
            pl.BlockSpec((nb, h), lambda i: (i, 0)),
            pl.BlockSpec((NUM_CORES, nb, h), lambda i: (0, i, 0)),
            pl.BlockSpec((nb, 1), lambda i: (i, 0)),
            pl.BlockSpec((h, out_dim), lambda i: (0, 0)),
            pl.BlockSpec((1, out_dim), lambda i: (0, 0)),
        ],
        out_specs=pl.BlockSpec((1, out_dim), lambda i: (0, 0)),
        out_shape=jax.ShapeDtypeStruct((1, out_dim), jnp.float32),
        scratch_shapes=[pltpu.VMEM((1, h), jnp.float32)],
    )(u, s_part, deg_col, W_out, b_out.reshape(1, out_dim))

    return out.reshape(out_dim)


# speedup vs baseline: 9.1268x; 1.1631x over previous
import functools

import jax
import jax.numpy as jnp
from jax import lax
from jax.experimental import pallas as pl
from jax.experimental.pallas import tpu as pltpu
from jax.experimental.pallas import tpu_sc as plsc

NUM_CORES = 2
NUM_SUBCORES = 16
NUM_TILES = NUM_CORES * NUM_SUBCORES
CH = 100
GRP = 10
ZB = 32
HR = 80
DBLK = 32


def _pre_body(ns_ref, w_in_ref, b_in_ref, w_self_ref, b_self_ref,
              w_nei_ref, b_nei_ref, u_ref, v_ref):
    x = jnp.dot(ns_ref[...], w_in_ref[...], preferred_element_type=jnp.float32)
    x = jnp.maximum(x + b_in_ref[...], 0.0)
    u = jnp.dot(x, w_self_ref[...], preferred_element_type=jnp.float32)
    u_ref[...] = u + b_self_ref[...] + b_nei_ref[...]
    v_ref[...] = jnp.dot(x, w_nei_ref[...], preferred_element_type=jnp.float32)


def _post_body(u_ref, s_ref, deg_ref, w_out_ref, b_out_ref, out_ref, acc_ref,
               *, inv_n):
    j = pl.program_id(0)

    @pl.when(j == 0)
    def _():
        acc_ref[...] = jnp.zeros_like(acc_ref)

    s = s_ref[0] + s_ref[1]
    agg = s / jnp.maximum(deg_ref[...], 1.0)
    hh = jnp.maximum(u_ref[...] + agg, 0.0)
    acc_ref[...] += jnp.sum(hh, axis=0, keepdims=True)

    @pl.when(j == pl.num_programs(0) - 1)
    def _():
        g = acc_ref[...] * inv_n
        out_ref[...] = (
            jnp.dot(g, w_out_ref[...], preferred_element_type=jnp.float32)
            + b_out_ref[...])


def _make_sc_segsum(npad, h, nch, ndst):
    prow = npad // NUM_SUBCORES
    assert prow % ZB == 0 and npad == HR * h
    mesh = plsc.VectorSubcoreMesh(
        core_axis_name="c", subcore_axis_name="s",
        num_cores=NUM_CORES, num_subcores=NUM_SUBCORES)

    @functools.partial(
        pl.kernel, mesh=mesh,
        compiler_params=pltpu.CompilerParams(needs_layout_passes=False),
        out_type=[
            jax.ShapeDtypeStruct((NUM_CORES, npad, h), jnp.float32),
            jax.ShapeDtypeStruct((NUM_CORES, NUM_SUBCORES, npad), jnp.float32),
        ],
        scratch_types=[
            pltpu.VMEM_SHARED((npad, h), jnp.float32),
            pltpu.VMEM((GRP, CH), jnp.int32),
            pltpu.VMEM((GRP, CH), jnp.int32),
            pltpu.VMEM((2, CH, h), jnp.float32),
            pltpu.VMEM((ZB, h), jnp.float32),
            pltpu.VMEM((npad,), jnp.float32),
            pltpu.VMEM((DBLK, 16), jnp.int32),
            pltpu.SemaphoreType.DMA,
            pltpu.SemaphoreType.DMA,
        ],
    )
    def sc_segsum(v_hbm, src_hbm, dst_hbm, dst16_hbm, zrows_hbm, zflat_hbm,
                  s_out, deg_out, accum, src_v, dst_v, rows2, zbuf,
                  hist, dst16_v, sem0, sem1):
        c = lax.axis_index("c")
        s = lax.axis_index("s")
        wid = c * NUM_SUBCORES + s
        base = s * prow
        sems = (sem0, sem1)

        pltpu.sync_copy(zrows_hbm, zbuf)

        @pl.loop(0, prow // ZB)
        def zblk(k):
            pltpu.sync_copy(zbuf, accum.at[pl.ds(base + k * ZB, ZB)])

        zf = zflat_hbm.shape[0]

        @pl.loop(0, npad // zf)
        def zhist(k):
            pltpu.sync_copy(zflat_hbm, hist.at[pl.ds(k * zf, zf)])

        plsc.subcore_barrier()

        @pl.loop(0, nch // GRP)
        def group(g):
            pltpu.sync_copy(src_hbm.at[wid, g], src_v)
            pltpu.sync_copy(dst_hbm.at[wid, g], dst_v)
            descs = [None, None]
            descs[0] = pltpu.async_copy(
                v_hbm.at[src_v.at[0]], rows2.at[0], sems[0])
            for r in range(GRP):
                b = r % 2
                descs[b].wait()
                if r + 1 < GRP:
                    nb = (r + 1) % 2
                    descs[nb] = pltpu.async_copy(
                        v_hbm.at[src_v.at[r + 1]], rows2.at[nb], sems[nb])
                pltpu.sync_copy(rows2.at[b], accum.at[dst_v.at[r]], add=True)

        ones16 = jnp.ones((16,), jnp.float32)

        @pl.loop(0, ndst // DBLK)
        def hblk(t):
            pltpu.sync_copy(dst16_hbm.at[wid, pl.ds(t * DBLK, DBLK)], dst16_v)
            for i in range(DBLK):
                plsc.addupdate_scatter(hist, [dst16_v[i]], ones16)

        plsc.subcore_barrier()

        @pl.loop(0, prow // ZB)
        def oblk(k):
            pltpu.sync_copy(accum.at[pl.ds(base + k * ZB, ZB)], zbuf)
            pltpu.sync_copy(zbuf, s_out.at[c, pl.ds(base + k * ZB, ZB)])

        pltpu.sync_copy(hist, deg_out.at[c, s])

    return sc_segsum


def kernel(node_scalar, edge_index, W_in, b_in, W_self, b_self, W_nei, b_nei,
           W_out, b_out):
    n, fin = node_scalar.shape
    h = W_in.shape[1]
    out_dim = W_out.shape[1]
    e = edge_index.shape[1]
    assert e % (NUM_TILES * CH) == 0
    nch = e // (NUM_TILES * CH)
    npad = HR * h
    assert npad >= n

    nb = 2000
    u, v = pl.pallas_call(
        _pre_body,
        grid=(n // nb,),
        in_specs=[
            pl.BlockSpec((nb, fin), lambda i: (i, 0)),
            pl.BlockSpec((fin, h), lambda i: (0, 0)),
            pl.BlockSpec((1, h), lambda i: (0, 0)),
            pl.BlockSpec((h, h), lambda i: (0, 0)),
            pl.BlockSpec((1, h), lambda i: (0, 0)),
            pl.BlockSpec((h, h), lambda i: (0, 0)),
            pl.BlockSpec((1, h), lambda i: (0, 0)),
        ],
        out_specs=[
            pl.BlockSpec((nb, h), lambda i: (i, 0)),
            pl.BlockSpec((nb, h), lambda i: (i, 0)),
        ],
        out_shape=[
            jax.ShapeDtypeStruct((n, h), jnp.float32),
            jax.ShapeDtypeStruct((n, h), jnp.float32),
        ],
    )(node_scalar, W_in, b_in.reshape(1, h), W_self, b_self.reshape(1, h),
      W_nei, b_nei.reshape(1, h))

    assert nch % GRP == 0
    src3 = edge_index[0].reshape(NUM_TILES, nch // GRP, GRP, CH)
    dst3 = edge_index[1].reshape(NUM_TILES, nch // GRP, GRP, CH)
    ept = -(-e // (NUM_TILES * DBLK * 16)) * DBLK * 16
    ndst = ept // 16
    pad = NUM_TILES * ept - e
    dst16 = jnp.concatenate(
        [edge_index[1], jnp.full((pad,), npad - 1, jnp.int32)]
    ).reshape(NUM_TILES, ndst, 16)
    zrows = jnp.zeros((ZB, h), jnp.float32)
    zflat = jnp.zeros((2048,), jnp.float32)
    s_part, deg_part = _make_sc_segsum(npad, h, nch, ndst)(
        v, src3, dst3, dst16, zrows, zflat)

    dsum = pl.pallas_call(
        lambda d_ref, o_ref: o_ref.__setitem__(
            (Ellipsis,), jnp.sum(d_ref[...], axis=0)),
        in_specs=[pl.BlockSpec((NUM_TILES, HR, h), lambda: (0, 0, 0))],
        out_specs=pl.BlockSpec((HR, h), lambda: (0, 0)),
        out_shape=jax.ShapeDtypeStruct((HR, h), jnp.float32),
    )(deg_part.reshape(NUM_TILES, HR, h))
    deg_col = dsum.reshape(npad, 1)

    out = pl.pallas_call(
        functools.partial(_post_body, inv_n=1.0 / n),
        grid=(n // nb,),
        in_specs=[
            pl.BlockSpec((nb, h), lambda i: (i, 0)),
            pl.BlockSpec((NUM_CORES, nb, h), lambda i: (0, i, 0)),
            pl.BlockSpec((nb, 1), lambda i: (i, 0)),
            pl.BlockSpec((h, out_dim), lambda i: (0, 0)),
            pl.BlockSpec((1, out_dim), lambda i: (0, 0)),
        ],
        out_specs=pl.BlockSpec((1, out_dim), lambda i: (0, 0)),
        out_shape=jax.ShapeDtypeStruct((1, out_dim), jnp.float32),
        scratch_shapes=[pltpu.VMEM((1, h), jnp.float32)],
    )(u, s_part, deg_col, W_out, b_out.reshape(1, out_dim))

    return out.reshape(out_dim)


# speedup vs baseline: 9.8002x; 1.0738x over previous
import functools

import jax
import jax.numpy as jnp
from jax import lax
from jax.experimental import pallas as pl
from jax.experimental.pallas import tpu as pltpu
from jax.experimental.pallas import tpu_sc as plsc

NUM_CORES = 2
NUM_SUBCORES = 16
NUM_TILES = NUM_CORES * NUM_SUBCORES
CH = 64
GRP = 10
NBUF = 3
ZB = 32
HR = 80
DBLK = 32


def _pre_body(ns_ref, w_in_ref, b_in_ref, w_self_ref, b_self_ref,
              w_nei_ref, b_nei_ref, u_ref, v_ref):
    x = jnp.dot(ns_ref[...], w_in_ref[...], preferred_element_type=jnp.float32)
    x = jnp.maximum(x + b_in_ref[...], 0.0)
    u = jnp.dot(x, w_self_ref[...], preferred_element_type=jnp.float32)
    u_ref[...] = u + b_self_ref[...] + b_nei_ref[...]
    v_ref[...] = jnp.dot(x, w_nei_ref[...], preferred_element_type=jnp.float32)


def _post_body(u_ref, s_ref, deg_ref, w_out_ref, b_out_ref, out_ref, acc_ref,
               *, inv_n):
    j = pl.program_id(0)

    @pl.when(j == 0)
    def _():
        acc_ref[...] = jnp.zeros_like(acc_ref)

    s = s_ref[0] + s_ref[1]
    agg = s / jnp.maximum(deg_ref[...], 1.0)
    hh = jnp.maximum(u_ref[...] + agg, 0.0)
    acc_ref[...] += jnp.sum(hh, axis=0, keepdims=True)

    @pl.when(j == pl.num_programs(0) - 1)
    def _():
        g = acc_ref[...] * inv_n
        out_ref[...] = (
            jnp.dot(g, w_out_ref[...], preferred_element_type=jnp.float32)
            + b_out_ref[...])


def _make_sc_segsum(npad, h, nch, ndst):
    prow = npad // NUM_SUBCORES
    assert prow % ZB == 0 and npad == HR * h
    mesh = plsc.VectorSubcoreMesh(
        core_axis_name="c", subcore_axis_name="s",
        num_cores=NUM_CORES, num_subcores=NUM_SUBCORES)

    @functools.partial(
        pl.kernel, mesh=mesh,
        compiler_params=pltpu.CompilerParams(needs_layout_passes=False),
        out_type=[
            jax.ShapeDtypeStruct((NUM_CORES, npad, h), jnp.float32),
            jax.ShapeDtypeStruct((NUM_CORES, NUM_SUBCORES, npad), jnp.float32),
        ],
        scratch_types=[
            pltpu.VMEM_SHARED((npad, h), jnp.float32),
            pltpu.VMEM((GRP, CH), jnp.int32),
            pltpu.VMEM((GRP, CH), jnp.int32),
            pltpu.VMEM((NBUF, CH, h), jnp.float32),
            pltpu.VMEM((ZB, h), jnp.float32),
            pltpu.VMEM((npad,), jnp.float32),
            pltpu.VMEM((DBLK, 16), jnp.int32),
            pltpu.SemaphoreType.DMA,
            pltpu.SemaphoreType.DMA,
            pltpu.SemaphoreType.DMA,
            pltpu.SemaphoreType.DMA,
            pltpu.SemaphoreType.DMA,
            pltpu.SemaphoreType.DMA,
        ],
    )
    def sc_segsum(v_hbm, src_hbm, dst_hbm, dst16_hbm, zrows_hbm, zflat_hbm,
                  s_out, deg_out, accum, src_v, dst_v, rows2, zbuf,
                  hist, dst16_v, gsem0, gsem1, gsem2, ssem0, ssem1, ssem2):
        c = lax.axis_index("c")
        s = lax.axis_index("s")
        wid = c * NUM_SUBCORES + s
        base = s * prow
        gsems = (gsem0, gsem1, gsem2)
        ssems = (ssem0, ssem1, ssem2)

        pltpu.sync_copy(zrows_hbm, zbuf)

        @pl.loop(0, prow // ZB)
        def zblk(k):
            pltpu.sync_copy(zbuf, accum.at[pl.ds(base + k * ZB, ZB)])

        zf = zflat_hbm.shape[0]

        @pl.loop(0, npad // zf)
        def zhist(k):
            pltpu.sync_copy(zflat_hbm, hist.at[pl.ds(k * zf, zf)])

        plsc.subcore_barrier()

        @pl.loop(0, nch // GRP)
        def group(g):
            pltpu.sync_copy(src_hbm.at[wid, g], src_v)
            pltpu.sync_copy(dst_hbm.at[wid, g], dst_v)
            descs_g = [None] * NBUF
            descs_s = [None] * NBUF
            descs_g[0] = pltpu.async_copy(
                v_hbm.at[src_v.at[0]], rows2.at[0], gsems[0])
            descs_g[1] = pltpu.async_copy(
                v_hbm.at[src_v.at[1]], rows2.at[1], gsems[1])
            for r in range(GRP):
                b = r % NBUF
                descs_g[b].wait()
                descs_s[b] = pltpu.async_copy(
                    rows2.at[b], accum.at[dst_v.at[r]], ssems[b], add=True)
                nxt = r + 2
                if nxt < GRP:
                    nb = nxt % NBUF
                    if descs_s[nb] is not None:
                        descs_s[nb].wait()
                    descs_g[nb] = pltpu.async_copy(
                        v_hbm.at[src_v.at[nxt]], rows2.at[nb], gsems[nb])
            for b in range(NBUF):
                if descs_s[b] is not None:
                    descs_s[b].wait()

        ones16 = jnp.ones((16,), jnp.float32)

        @pl.loop(0, ndst // DBLK)
        def hblk(t):
            pltpu.sync_copy(dst16_hbm.at[wid, pl.ds(t * DBLK, DBLK)], dst16_v)
            for i in range(DBLK):
                plsc.addupdate_scatter(hist, [dst16_v[i]], ones16)

        plsc.subcore_barrier()

        @pl.loop(0, prow // ZB)
        def oblk(k):
            pltpu.sync_copy(accum.at[pl.ds(base + k * ZB, ZB)], zbuf)
            pltpu.sync_copy(zbuf, s_out.at[c, pl.ds(base + k * ZB, ZB)])

        pltpu.sync_copy(hist, deg_out.at[c, s])

    return sc_segsum


def kernel(node_scalar, edge_index, W_in, b_in, W_self, b_self, W_nei, b_nei,
           W_out, b_out):
    n, fin = node_scalar.shape
    h = W_in.shape[1]
    out_dim = W_out.shape[1]
    e = edge_index.shape[1]
    npad = HR * h
    assert npad >= n
    ept = -(-e // (NUM_TILES * CH * GRP)) * CH * GRP
    epad = NUM_TILES * ept - e
    nch = ept // CH

    nb = 2000
    u, v = pl.pallas_call(
        _pre_body,
        grid=(n // nb,),
        in_specs=[
            pl.BlockSpec((nb, fin), lambda i: (i, 0)),
            pl.BlockSpec((fin, h), lambda i: (0, 0)),
            pl.BlockSpec((1, h), lambda i: (0, 0)),
            pl.BlockSpec((h, h), lambda i: (0, 0)),
            pl.BlockSpec((1, h), lambda i: (0, 0)),
            pl.BlockSpec((h, h), lambda i: (0, 0)),
            pl.BlockSpec((1, h), lambda i: (0, 0)),
        ],
        out_specs=[
            pl.BlockSpec((nb, h), lambda i: (i, 0)),
            pl.BlockSpec((nb, h), lambda i: (i, 0)),
        ],
        out_shape=[
            jax.ShapeDtypeStruct((n, h), jnp.float32),
            jax.ShapeDtypeStruct((n, h), jnp.float32),
        ],
    )(node_scalar, W_in, b_in.reshape(1, h), W_self, b_self.reshape(1, h),
      W_nei, b_nei.reshape(1, h))

    assert nch % GRP == 0
    pad_src = (jnp.arange(epad, dtype=jnp.int32) * 97) % n
    pad_dst = n + (jnp.arange(epad, dtype=jnp.int32) % (npad - n))
    src3 = jnp.concatenate([edge_index[0], pad_src]).reshape(
        NUM_TILES, nch // GRP, GRP, CH)
    dst3 = jnp.concatenate([edge_index[1], pad_dst]).reshape(
        NUM_TILES, nch // GRP, GRP, CH)
    ept = -(-e // (NUM_TILES * DBLK * 16)) * DBLK * 16
    ndst = ept // 16
    pad = NUM_TILES * ept - e
    dst16 = jnp.concatenate(
        [edge_index[1], jnp.full((pad,), npad - 1, jnp.int32)]
    ).reshape(NUM_TILES, ndst, 16)
    zrows = jnp.zeros((ZB, h), jnp.float32)
    zflat = jnp.zeros((2048,), jnp.float32)
    s_part, deg_part = _make_sc_segsum(npad, h, nch, ndst)(
        v, src3, dst3, dst16, zrows, zflat)

    dsum = pl.pallas_call(
        lambda d_ref, o_ref: o_ref.__setitem__(
            (Ellipsis,), jnp.sum(d_ref[...], axis=0)),
        in_specs=[pl.BlockSpec((NUM_TILES, HR, h), lambda: (0, 0, 0))],
        out_specs=pl.BlockSpec((HR, h), lambda: (0, 0)),
        out_shape=jax.ShapeDtypeStruct((HR, h), jnp.float32),
    )(deg_part.reshape(NUM_TILES, HR, h))
    deg_col = dsum.reshape(npad, 1)

    out = pl.pallas_call(
        functools.partial(_post_body, inv_n=1.0 / n),
        grid=(n // nb,),
        in_specs=[
            pl.BlockSpec((nb, h), lambda i: (i, 0)),
            pl.BlockSpec((NUM_CORES, nb, h), lambda i: (0, i, 0)),
            pl.BlockSpec((nb, 1), lambda i: (i, 0)),
            pl.BlockSpec((h, out_dim), lambda i: (0, 0)),
            pl.BlockSpec((1, out_dim), lambda i: (0, 0)),
        ],
        out_specs=pl.BlockSpec((1, out_dim), lambda i: (0, 0)),
        out_shape=jax.ShapeDtypeStruct((1, out_dim), jnp.float32),
        scratch_shapes=[pltpu.VMEM((1, h), jnp.float32)],
    )(u, s_part, deg_col, W_out, b_out.reshape(1, out_dim))

    return out.reshape(out_dim)


# speedup vs baseline: 10.4984x; 1.0712x over previous
import functools

import jax
import jax.numpy as jnp
from jax import lax
from jax.experimental import pallas as pl
from jax.experimental.pallas import tpu as pltpu
from jax.experimental.pallas import tpu_sc as plsc

NUM_CORES = 2
NUM_SUBCORES = 16
NUM_TILES = NUM_CORES * NUM_SUBCORES
CH = 64
GRP = 20
NBUF = 3
ZB = 32
HR = 80
DBLK = 32


def _pre_body(ns_ref, w_in_ref, b_in_ref, w_self_ref, b_self_ref,
              w_nei_ref, b_nei_ref, u_ref, v_ref):
    x = jnp.dot(ns_ref[...], w_in_ref[...], preferred_element_type=jnp.float32)
    x = jnp.maximum(x + b_in_ref[...], 0.0)
    u = jnp.dot(x, w_self_ref[...], preferred_element_type=jnp.float32)
    u_ref[...] = u + b_self_ref[...] + b_nei_ref[...]
    v_ref[...] = jnp.dot(x, w_nei_ref[...], preferred_element_type=jnp.float32)


def _post_body(u_ref, s_ref, deg_ref, w_out_ref, b_out_ref, out_ref, acc_ref,
               *, inv_n):
    j = pl.program_id(0)

    @pl.when(j == 0)
    def _():
        acc_ref[...] = jnp.zeros_like(acc_ref)

    s = s_ref[0] + s_ref[1]
    agg = s / jnp.maximum(deg_ref[...], 1.0)
    hh = jnp.maximum(u_ref[...] + agg, 0.0)
    acc_ref[...] += jnp.sum(hh, axis=0, keepdims=True)

    @pl.when(j == pl.num_programs(0) - 1)
    def _():
        g = acc_ref[...] * inv_n
        out_ref[...] = (
            jnp.dot(g, w_out_ref[...], preferred_element_type=jnp.float32)
            + b_out_ref[...])


def _make_sc_segsum(npad, h, nch, ndst):
    prow = npad // NUM_SUBCORES
    assert prow % ZB == 0 and npad == HR * h
    mesh = plsc.VectorSubcoreMesh(
        core_axis_name="c", subcore_axis_name="s",
        num_cores=NUM_CORES, num_subcores=NUM_SUBCORES)

    @functools.partial(
        pl.kernel, mesh=mesh,
        compiler_params=pltpu.CompilerParams(needs_layout_passes=False),
        out_type=[
            jax.ShapeDtypeStruct((NUM_CORES, npad, h), jnp.float32),
            jax.ShapeDtypeStruct((NUM_CORES, NUM_SUBCORES, npad), jnp.float32),
        ],
        scratch_types=[
            pltpu.VMEM_SHARED((npad, h), jnp.float32),
            pltpu.VMEM((GRP, CH), jnp.int32),
            pltpu.VMEM((GRP, CH), jnp.int32),
            pltpu.VMEM((NBUF, CH, h), jnp.float32),
            pltpu.VMEM((ZB, h), jnp.float32),
            pltpu.VMEM((npad,), jnp.float32),
            pltpu.VMEM((DBLK, 16), jnp.int32),
            pltpu.SemaphoreType.DMA,
            pltpu.SemaphoreType.DMA,
            pltpu.SemaphoreType.DMA,
            pltpu.SemaphoreType.DMA,
            pltpu.SemaphoreType.DMA,
            pltpu.SemaphoreType.DMA,
        ],
    )
    def sc_segsum(v_hbm, src_hbm, dst_hbm, dst16_hbm, zrows_hbm, zflat_hbm,
                  s_out, deg_out, accum, src_v, dst_v, rows2, zbuf,
                  hist, dst16_v, gsem0, gsem1, gsem2, ssem0, ssem1, ssem2):
        c = lax.axis_index("c")
        s = lax.axis_index("s")
        wid = c * NUM_SUBCORES + s
        base = s * prow
        gsems = (gsem0, gsem1, gsem2)
        ssems = (ssem0, ssem1, ssem2)

        pltpu.sync_copy(zrows_hbm, zbuf)

        @pl.loop(0, prow // ZB)
        def zblk(k):
            pltpu.sync_copy(zbuf, accum.at[pl.ds(base + k * ZB, ZB)])

        zf = zflat_hbm.shape[0]

        @pl.loop(0, npad // zf)
        def zhist(k):
            pltpu.sync_copy(zflat_hbm, hist.at[pl.ds(k * zf, zf)])

        plsc.subcore_barrier()

        @pl.loop(0, nch // GRP)
        def group(g):
            pltpu.sync_copy(src_hbm.at[wid, g], src_v)
            pltpu.sync_copy(dst_hbm.at[wid, g], dst_v)
            descs_g = [None] * NBUF
            descs_s = [None] * NBUF
            descs_g[0] = pltpu.async_copy(
                v_hbm.at[src_v.at[0]], rows2.at[0], gsems[0])
            descs_g[1] = pltpu.async_copy(
                v_hbm.at[src_v.at[1]], rows2.at[1], gsems[1])
            for r in range(GRP):
                b = r % NBUF
                descs_g[b].wait()
                descs_s[b] = pltpu.async_copy(
                    rows2.at[b], accum.at[dst_v.at[r]], ssems[b], add=True)
                nxt = r + 2
                if nxt < GRP:
                    nb = nxt % NBUF
                    if descs_s[nb] is not None:
                        descs_s[nb].wait()
                    descs_g[nb] = pltpu.async_copy(
                        v_hbm.at[src_v.at[nxt]], rows2.at[nb], gsems[nb])
            for b in range(NBUF):
                if descs_s[b] is not None:
                    descs_s[b].wait()

        ones16 = jnp.ones((16,), jnp.float32)

        @pl.loop(0, ndst // DBLK)
        def hblk(t):
            pltpu.sync_copy(dst16_hbm.at[wid, pl.ds(t * DBLK, DBLK)], dst16_v)
            for i in range(DBLK):
                plsc.addupdate_scatter(hist, [dst16_v[i]], ones16)

        plsc.subcore_barrier()

        @pl.loop(0, prow // ZB)
        def oblk(k):
            pltpu.sync_copy(accum.at[pl.ds(base + k * ZB, ZB)], zbuf)
            pltpu.sync_copy(zbuf, s_out.at[c, pl.ds(base + k * ZB, ZB)])

        pltpu.sync_copy(hist, deg_out.at[c, s])

    return sc_segsum


def kernel(node_scalar, edge_index, W_in, b_in, W_self, b_self, W_nei, b_nei,
           W_out, b_out):
    n, fin = node_scalar.shape
    h = W_in.shape[1]
    out_dim = W_out.shape[1]
    e = edge_index.shape[1]
    npad = HR * h
    assert npad >= n
    ept = -(-e // (NUM_TILES * CH * GRP)) * CH * GRP
    epad = NUM_TILES * ept - e
    nch = ept // CH

    nb = 2000
    u, v = pl.pallas_call(
        _pre_body,
        grid=(n // nb,),
        in_specs=[
            pl.BlockSpec((nb, fin), lambda i: (i, 0)),
            pl.BlockSpec((fin, h), lambda i: (0, 0)),
            pl.BlockSpec((1, h), lambda i: (0, 0)),
            pl.BlockSpec((h, h), lambda i: (0, 0)),
            pl.BlockSpec((1, h), lambda i: (0, 0)),
            pl.BlockSpec((h, h), lambda i: (0, 0)),
            pl.BlockSpec((1, h), lambda i: (0, 0)),
        ],
        out_specs=[
            pl.BlockSpec((nb, h), lambda i: (i, 0)),
            pl.BlockSpec((nb, h), lambda i: (i, 0)),
        ],
        out_shape=[
            jax.ShapeDtypeStruct((n, h), jnp.float32),
            jax.ShapeDtypeStruct((n, h), jnp.float32),
        ],
    )(node_scalar, W_in, b_in.reshape(1, h), W_self, b_self.reshape(1, h),
      W_nei, b_nei.reshape(1, h))

    assert nch % GRP == 0
    pad_src = (jnp.arange(epad, dtype=jnp.int32) * 97) % n
    pad_dst = n + (jnp.arange(epad, dtype=jnp.int32) % (npad - n))
    src3 = jnp.concatenate([edge_index[0], pad_src]).reshape(
        NUM_TILES, nch // GRP, GRP, CH)
    dst3 = jnp.concatenate([edge_index[1], pad_dst]).reshape(
        NUM_TILES, nch // GRP, GRP, CH)
    ndst = ept // 16
    dst16 = dst3.reshape(NUM_TILES, ndst, 16)
    zrows = jnp.zeros((ZB, h), jnp.float32)
    zflat = jnp.zeros((2048,), jnp.float32)
    s_part, deg_part = _make_sc_segsum(npad, h, nch, ndst)(
        v, src3, dst3, dst16, zrows, zflat)

    dsum = pl.pallas_call(
        lambda d_ref, o_ref: o_ref.__setitem__(
            (Ellipsis,), jnp.sum(d_ref[...], axis=0)),
        in_specs=[pl.BlockSpec((NUM_TILES, HR, h), lambda: (0, 0, 0))],
        out_specs=pl.BlockSpec((HR, h), lambda: (0, 0)),
        out_shape=jax.ShapeDtypeStruct((HR, h), jnp.float32),
    )(deg_part.reshape(NUM_TILES, HR, h))
    deg_col = dsum.reshape(npad, 1)

    out = pl.pallas_call(
        functools.partial(_post_body, inv_n=1.0 / n),
        grid=(n // nb,),
        in_specs=[
            pl.BlockSpec((nb, h), lambda i: (i, 0)),
            pl.BlockSpec((NUM_CORES, nb, h), lambda i: (0, i, 0)),
            pl.BlockSpec((nb, 1), lambda i: (i, 0)),
            pl.BlockSpec((h, out_dim), lambda i: (0, 0)),
            pl.BlockSpec((1, out_dim), lambda i: (0, 0)),
        ],
        out_specs=pl.BlockSpec((1, out_dim), lambda i: (0, 0)),
        out_shape=jax.ShapeDtypeStruct((1, out_dim), jnp.float32),
        scratch_shapes=[pltpu.VMEM((1, h), jnp.float32)],
    )(u, s_part, deg_col, W_out, b_out.reshape(1, out_dim))

    return out.reshape(out_dim)


# speedup vs baseline: 10.5860x; 1.0083x over previous
import functools

import jax
import jax.numpy as jnp
from jax import lax
from jax.experimental import pallas as pl
from jax.experimental.pallas import tpu as pltpu
from jax.experimental.pallas import tpu_sc as plsc

NUM_CORES = 2
NUM_SUBCORES = 16
NUM_TILES = NUM_CORES * NUM_SUBCORES
CH = 64
GRP = 32
NBUF = 3
ZB = 16
HR = 80
DBLK = 32


def _pre_body(ns_ref, w_in_ref, b_in_ref, w_self_ref, b_self_ref,
              w_nei_ref, b_nei_ref, u_ref, v_ref):
    x = jnp.dot(ns_ref[...], w_in_ref[...], preferred_element_type=jnp.float32)
    x = jnp.maximum(x + b_in_ref[...], 0.0)
    u = jnp.dot(x, w_self_ref[...], preferred_element_type=jnp.float32)
    u_ref[...] = u + b_self_ref[...] + b_nei_ref[...]
    v_ref[...] = jnp.dot(x, w_nei_ref[...], preferred_element_type=jnp.float32)


def _post_body(u_ref, s_ref, deg_ref, w_out_ref, b_out_ref, out_ref, acc_ref,
               *, inv_n):
    j = pl.program_id(0)

    @pl.when(j == 0)
    def _():
        acc_ref[...] = jnp.zeros_like(acc_ref)

    s = s_ref[0] + s_ref[1]
    agg = s / jnp.maximum(deg_ref[...], 1.0)
    hh = jnp.maximum(u_ref[...] + agg, 0.0)
    acc_ref[...] += jnp.sum(hh, axis=0, keepdims=True)

    @pl.when(j == pl.num_programs(0) - 1)
    def _():
        g = acc_ref[...] * inv_n
        out_ref[...] = (
            jnp.dot(g, w_out_ref[...], preferred_element_type=jnp.float32)
            + b_out_ref[...])


def _make_sc_segsum(npad, h, nch, ndst):
    prow = npad // NUM_SUBCORES
    assert prow % ZB == 0 and npad == HR * h
    mesh = plsc.VectorSubcoreMesh(
        core_axis_name="c", subcore_axis_name="s",
        num_cores=NUM_CORES, num_subcores=NUM_SUBCORES)

    @functools.partial(
        pl.kernel, mesh=mesh,
        compiler_params=pltpu.CompilerParams(needs_layout_passes=False),
        out_type=[
            jax.ShapeDtypeStruct((NUM_CORES, npad, h), jnp.float32),
            jax.ShapeDtypeStruct((NUM_CORES, NUM_SUBCORES, npad), jnp.float32),
        ],
        scratch_types=[
            pltpu.VMEM_SHARED((npad, h), jnp.float32),
            pltpu.VMEM((GRP, CH), jnp.int32),
            pltpu.VMEM((GRP, CH), jnp.int32),
            pltpu.VMEM((NBUF, CH, h), jnp.float32),
            pltpu.VMEM((ZB, h), jnp.float32),
            pltpu.VMEM((npad,), jnp.float32),
            pltpu.VMEM((DBLK, 16), jnp.int32),
            pltpu.SemaphoreType.DMA,
            pltpu.SemaphoreType.DMA,
            pltpu.SemaphoreType.DMA,
            pltpu.SemaphoreType.DMA,
            pltpu.SemaphoreType.DMA,
            pltpu.SemaphoreType.DMA,
        ],
    )
    def sc_segsum(v_hbm, src_hbm, dst_hbm, dst16_hbm, zrows_hbm, zflat_hbm,
                  s_out, deg_out, accum, src_v, dst_v, rows2, zbuf,
                  hist, dst16_v, gsem0, gsem1, gsem2, ssem0, ssem1, ssem2):
        c = lax.axis_index("c")
        s = lax.axis_index("s")
        wid = c * NUM_SUBCORES + s
        base = s * prow
        gsems = (gsem0, gsem1, gsem2)
        ssems = (ssem0, ssem1, ssem2)

        pltpu.sync_copy(zrows_hbm, zbuf)

        @pl.loop(0, prow // ZB)
        def zblk(k):
            pltpu.sync_copy(zbuf, accum.at[pl.ds(base + k * ZB, ZB)])

        zf = zflat_hbm.shape[0]

        @pl.loop(0, npad // zf)
        def zhist(k):
            pltpu.sync_copy(zflat_hbm, hist.at[pl.ds(k * zf, zf)])

        plsc.subcore_barrier()

        @pl.loop(0, nch // GRP)
        def group(g):
            pltpu.sync_copy(src_hbm.at[wid, g], src_v)
            pltpu.sync_copy(dst_hbm.at[wid, g], dst_v)
            descs_g = [None] * NBUF
            descs_s = [None] * NBUF
            descs_g[0] = pltpu.async_copy(
                v_hbm.at[src_v.at[0]], rows2.at[0], gsems[0])
            descs_g[1] = pltpu.async_copy(
                v_hbm.at[src_v.at[1]], rows2.at[1], gsems[1])
            for r in range(GRP):
                b = r % NBUF
                descs_g[b].wait()
                descs_s[b] = pltpu.async_copy(
                    rows2.at[b], accum.at[dst_v.at[r]], ssems[b], add=True)
                nxt = r + 2
                if nxt < GRP:
                    nb = nxt % NBUF
                    if descs_s[nb] is not None:
                        descs_s[nb].wait()
                    descs_g[nb] = pltpu.async_copy(
                        v_hbm.at[src_v.at[nxt]], rows2.at[nb], gsems[nb])
            for b in range(NBUF):
                if descs_s[b] is not None:
                    descs_s[b].wait()

        ones16 = jnp.ones((16,), jnp.float32)

        @pl.loop(0, ndst // DBLK)
        def hblk(t):
            pltpu.sync_copy(dst16_hbm.at[wid, pl.ds(t * DBLK, DBLK)], dst16_v)
            for i in range(DBLK):
                plsc.addupdate_scatter(hist, [dst16_v[i]], ones16)

        plsc.subcore_barrier()

        @pl.loop(0, prow // ZB)
        def oblk(k):
            pltpu.sync_copy(accum.at[pl.ds(base + k * ZB, ZB)], zbuf)
            pltpu.sync_copy(zbuf, s_out.at[c, pl.ds(base + k * ZB, ZB)])

        pltpu.sync_copy(hist, deg_out.at[c, s])

    return sc_segsum


def kernel(node_scalar, edge_index, W_in, b_in, W_self, b_self, W_nei, b_nei,
           W_out, b_out):
    n, fin = node_scalar.shape
    h = W_in.shape[1]
    out_dim = W_out.shape[1]
    e = edge_index.shape[1]
    npad = HR * h
    assert npad >= n
    ept = -(-e // (NUM_TILES * CH * GRP)) * CH * GRP
    epad = NUM_TILES * ept - e
    nch = ept // CH

    nb = 2000
    u, v = pl.pallas_call(
        _pre_body,
        grid=(n // nb,),
        in_specs=[
            pl.BlockSpec((nb, fin), lambda i: (i, 0)),
            pl.BlockSpec((fin, h), lambda i: (0, 0)),
            pl.BlockSpec((1, h), lambda i: (0, 0)),
            pl.BlockSpec((h, h), lambda i: (0, 0)),
            pl.BlockSpec((1, h), lambda i: (0, 0)),
            pl.BlockSpec((h, h), lambda i: (0, 0)),
            pl.BlockSpec((1, h), lambda i: (0, 0)),
        ],
        out_specs=[
            pl.BlockSpec((nb, h), lambda i: (i, 0)),
            pl.BlockSpec((nb, h), lambda i: (i, 0)),
        ],
        out_shape=[
            jax.ShapeDtypeStruct((n, h), jnp.float32),
            jax.ShapeDtypeStruct((n, h), jnp.float32),
        ],
    )(node_scalar, W_in, b_in.reshape(1, h), W_self, b_self.reshape(1, h),
      W_nei, b_nei.reshape(1, h))

    assert nch % GRP == 0
    pad_src = (jnp.arange(epad, dtype=jnp.int32) * 97) % n
    pad_dst = n + (jnp.arange(epad, dtype=jnp.int32) % (npad - n))
    src3 = jnp.concatenate([edge_index[0], pad_src]).reshape(
        NUM_TILES, nch // GRP, GRP, CH)
    dst3 = jnp.concatenate([edge_index[1], pad_dst]).reshape(
        NUM_TILES, nch // GRP, GRP, CH)
    ndst = ept // 16
    dst16 = dst3.reshape(NUM_TILES, ndst, 16)
    zrows = jnp.zeros((ZB, h), jnp.float32)
    zflat = jnp.zeros((2048,), jnp.float32)
    s_part, deg_part = _make_sc_segsum(npad, h, nch, ndst)(
        v, src3, dst3, dst16, zrows, zflat)

    dsum = pl.pallas_call(
        lambda d_ref, o_ref: o_ref.__setitem__(
            (Ellipsis,), jnp.sum(d_ref[...], axis=0)),
        in_specs=[pl.BlockSpec((NUM_TILES, HR, h), lambda: (0, 0, 0))],
        out_specs=pl.BlockSpec((HR, h), lambda: (0, 0)),
        out_shape=jax.ShapeDtypeStruct((HR, h), jnp.float32),
    )(deg_part.reshape(NUM_TILES, HR, h))
    deg_col = dsum.reshape(npad, 1)

    out = pl.pallas_call(
        functools.partial(_post_body, inv_n=1.0 / n),
        grid=(n // nb,),
        in_specs=[
            pl.BlockSpec((nb, h), lambda i: (i, 0)),
            pl.BlockSpec((NUM_CORES, nb, h), lambda i: (0, i, 0)),
            pl.BlockSpec((nb, 1), lambda i: (i, 0)),
            pl.BlockSpec((h, out_dim), lambda i: (0, 0)),
            pl.BlockSpec((1, out_dim), lambda i: (0, 0)),
        ],
        out_specs=pl.BlockSpec((1, out_dim), lambda i: (0, 0)),
        out_shape=jax.ShapeDtypeStruct((1, out_dim), jnp.float32),
        scratch_shapes=[pltpu.VMEM((1, h), jnp.float32)],
    )(u, s_part, deg_col, W_out, b_out.reshape(1, out_dim))

    return out.reshape(out_dim)
